# baseline (speedup 1.0000x reference)
import functools

import jax
import jax.numpy as jnp
from jax import lax
from jax.experimental import pallas as pl
from jax.experimental.pallas import tpu as pltpu
from jax.experimental.pallas import tpu_sc as plsc

BATCH = 16384
EMBED_DIM = 64
NUMERIC_DIM = 16
OUT_DIM = 2 * EMBED_DIM + NUMERIC_DIM

_info = plsc.get_sparse_core_info()
_NC, _NS, _NL = _info.num_cores, _info.num_subcores, _info.num_lanes
_NW = _NC * _NS
_BPW = BATCH // _NW
_NG = EMBED_DIM // _NL


@functools.partial(
    pl.kernel,
    mesh=plsc.VectorSubcoreMesh(core_axis_name="c", subcore_axis_name="s"),
    out_type=jax.ShapeDtypeStruct((BATCH, OUT_DIM), jnp.float32),
    scratch_types=[
        pltpu.VMEM((_BPW,), jnp.int32),
        pltpu.VMEM((_BPW,), jnp.int32),
        pltpu.VMEM((2, EMBED_DIM), jnp.float32),
        pltpu.VMEM((_BPW, EMBED_DIM), jnp.float32),
        pltpu.VMEM((_BPW, EMBED_DIM), jnp.float32),
        pltpu.VMEM((_BPW, NUMERIC_DIM), jnp.float32),
        pltpu.SemaphoreType.DMA,
    ],
    compiler_params=pltpu.CompilerParams(use_tc_tiling_on_sc=False,
                                         needs_layout_passes=False),
)
def _encode(sex_hbm, site_hbm, numeric_hbm, sex_table_hbm, site_table_hbm,
            out_hbm, sex_idx, site_idx, tab_v, sex_rows, site_rows, num_v,
            sem):
    wid = lax.axis_index("s") * _NC + lax.axis_index("c")
    base = wid * _BPW
    pltpu.sync_copy(sex_hbm.at[pl.ds(base, _BPW)], sex_idx)
    pltpu.sync_copy(site_hbm.at[pl.ds(base, _BPW)], site_idx)
    pltpu.sync_copy(sex_table_hbm, tab_v)
    g_site = pltpu.async_copy(site_table_hbm.at[site_idx], site_rows, sem)
    g_num = pltpu.async_copy(numeric_hbm.at[pl.ds(base, _BPW)], num_v, sem)

    r0 = [tab_v[0, pl.ds(g * _NL, _NL)] for g in range(_NG)]
    r1 = [tab_v[1, pl.ds(g * _NL, _NL)] for g in range(_NG)]
    zero = jnp.zeros((_NL,), jnp.int32)

    def row_body(i, carry):
        sv = plsc.load_gather(sex_idx, [jnp.full((_NL,), i, jnp.int32)])
        m = sv == zero
        for g in range(_NG):
            sex_rows[i, pl.ds(g * _NL, _NL)] = jnp.where(m, r0[g], r1[g])
        return carry

    lax.fori_loop(0, _BPW, row_body, 0)

    g_site.wait()
    g_num.wait()
    pltpu.sync_copy(sex_rows,
                    out_hbm.at[pl.ds(base, _BPW), pl.ds(0, EMBED_DIM)])
    pltpu.sync_copy(site_rows,
                    out_hbm.at[pl.ds(base, _BPW), pl.ds(EMBED_DIM, EMBED_DIM)])
    pltpu.sync_copy(num_v,
                    out_hbm.at[pl.ds(base, _BPW), pl.ds(2 * EMBED_DIM, NUMERIC_DIM)])


def kernel(sex, site, numeric, sex_table, site_table):
    return _encode(sex, site, numeric, sex_table, site_table)


# speedup vs baseline: 1.2347x; 1.2347x over previous
import functools

import jax
import jax.numpy as jnp
from jax import lax
from jax.experimental import pallas as pl
from jax.experimental.pallas import tpu as pltpu
from jax.experimental.pallas import tpu_sc as plsc

BATCH = 16384
EMBED_DIM = 64
NUMERIC_DIM = 16
OUT_DIM = 2 * EMBED_DIM + NUMERIC_DIM
EMB2 = 2 * EMBED_DIM

_info = plsc.get_sparse_core_info()
_NC, _NS, _NL = _info.num_cores, _info.num_subcores, _info.num_lanes
_NW = _NC * _NS
_BPW = BATCH // _NW
_NG = EMBED_DIM // _NL


@functools.partial(
    pl.kernel,
    mesh=plsc.VectorSubcoreMesh(core_axis_name="c", subcore_axis_name="s"),
    out_type=jax.ShapeDtypeStruct((BATCH, EMB2), jnp.float32),
    scratch_types=[
        pltpu.VMEM((_BPW,), jnp.int32),
        pltpu.VMEM((_BPW,), jnp.int32),
        pltpu.VMEM((EMB2,), jnp.float32),
        pltpu.VMEM((_BPW, EMBED_DIM), jnp.float32),
        pltpu.VMEM((_BPW, EMBED_DIM), jnp.float32),
        pltpu.SemaphoreType.DMA,
    ],
    compiler_params=pltpu.CompilerParams(use_tc_tiling_on_sc=False,
                                         needs_layout_passes=False),
)
def _embed(sex_hbm, site_hbm, sex_table_hbm, site_table_hbm,
           emb_hbm, sex_idx, site_idx, tab_v, sex_rows, site_rows, sem):
    wid = lax.axis_index("s") * _NC + lax.axis_index("c")
    base = wid * _BPW
    pltpu.sync_copy(sex_hbm.at[pl.ds(base, _BPW)], sex_idx)
    pltpu.sync_copy(site_hbm.at[pl.ds(base, _BPW)], site_idx)
    pltpu.sync_copy(sex_table_hbm, tab_v)
    g_site = pltpu.async_copy(site_table_hbm.at[site_idx], site_rows, sem)

    r0 = [tab_v[pl.ds(g * _NL, _NL)] for g in range(_NG)]
    r1 = [tab_v[pl.ds(EMBED_DIM + g * _NL, _NL)] for g in range(_NG)]
    zero = jnp.zeros((_NL,), jnp.int32)

    def row_body(i, carry):
        sv = plsc.load_gather(sex_idx, [jnp.full((_NL,), i, jnp.int32)])
        m = sv == zero
        for g in range(_NG):
            sex_rows[i, pl.ds(g * _NL, _NL)] = jnp.where(m, r0[g], r1[g])
        return carry

    lax.fori_loop(0, _BPW, row_body, 0)

    g_site.wait()
    pltpu.sync_copy(sex_rows,
                    emb_hbm.at[pl.ds(base, _BPW), pl.ds(0, EMBED_DIM)])
    pltpu.sync_copy(site_rows,
                    emb_hbm.at[pl.ds(base, _BPW), pl.ds(EMBED_DIM, EMBED_DIM)])


_TC_BLOCK = 2048


def _concat_body(emb_ref, num_ref, out_ref):
    out_ref[...] = jnp.concatenate([emb_ref[...], num_ref[...]], axis=1)


_concat = pl.pallas_call(
    _concat_body,
    grid=(BATCH // _TC_BLOCK,),
    in_specs=[
        pl.BlockSpec((_TC_BLOCK, EMB2), lambda i: (i, 0)),
        pl.BlockSpec((_TC_BLOCK, NUMERIC_DIM), lambda i: (i, 0)),
    ],
    out_specs=pl.BlockSpec((_TC_BLOCK, OUT_DIM), lambda i: (i, 0)),
    out_shape=jax.ShapeDtypeStruct((BATCH, OUT_DIM), jnp.float32),
)


def kernel(sex, site, numeric, sex_table, site_table):
    emb = _embed(sex, site, sex_table.reshape(-1), site_table)
    return _concat(emb, numeric)
